# baseline (speedup 1.0000x reference)
import functools

import jax
import jax.numpy as jnp
from jax import lax
from jax.experimental import pallas as pl
from jax.experimental.pallas import tpu as pltpu
from jax.experimental.pallas import tpu_sc as plsc

N_NODES = 10000
N_EDGES = 160000
D_LAT = 1024
BATCH = 8

NW = 32
EC = N_EDGES // NW
CHUNK = 128
ECP = 5120
NCH = ECP // CHUNK
ACC_ROWS = 10016
DUMMY = N_NODES

_mesh = lambda: plsc.VectorSubcoreMesh(core_axis_name="c", subcore_axis_name="s")


NTOT = 2 * NCH


def _agg_body(nsl, ones_mode, hp_hbm, src2_hbm, dst2_hbm, out_hbm,
              buf_a, buf_b, zbuf, srcall, didx_all, acc,
              sem_a, sem_b, sem_sa, sem_sb):
    c = lax.axis_index("c")
    s = lax.axis_index("s")

    def zz(i, _):
        zbuf[i // 8, pl.ds((i % 8) * 16, 16)] = jnp.zeros((16,), jnp.float32)
        return 0
    lax.fori_loop(0, 16 * 128 // 16, zz, 0)
    if ones_mode:
        def oo(i, _):
            buf_a[i // 8, pl.ds((i % 8) * 16, 16)] = (
                jnp.ones((16,), jnp.float32))
            return 0
        lax.fori_loop(0, CHUNK * 128 // 16, oo, 0)

    nzc = ACC_ROWS // 16
    for pp in range(nsl // 2):
        p = 2 * pp + c

        def zc(j, _):
            idx = s + j * 16

            @pl.when(idx < nzc)
            def _():
                pltpu.sync_copy(zbuf, acc.at[pl.ds(idx * 16, 16)])
            return 0
        lax.fori_loop(0, (nzc + 15) // 16, zc, 0)
        plsc.subcore_barrier()

        for half in range(2):
            eb = s + 16 * half
            pltpu.sync_copy(dst2_hbm.at[pl.ds(eb * NCH, NCH)], didx_all)
            if ones_mode:
                def chunk_body(r, _):
                    pltpu.sync_copy(buf_a, acc.at[didx_all.at[r]], add=True)
                    return 0
                lax.fori_loop(0, NCH, chunk_body, 0)
            else:
                pltpu.sync_copy(src2_hbm.at[pl.ds(eb * NCH, NCH)], srcall)
                off = p * N_NODES

                def ao(i, _):
                    srcall[i // 8, pl.ds((i % 8) * 16, 16)] = (
                        srcall[i // 8, pl.ds((i % 8) * 16, 16)] + off)
                    return 0
                lax.fori_loop(0, NCH * 8, ao, 0)

                pltpu.async_copy(hp_hbm.at[srcall.at[0]], buf_a, sem_a)

                def ping(r, buf_x, sem_gx, sem_sx, buf_y, sem_gy, sem_sy):
                    pltpu.make_async_copy(
                        hp_hbm.at[srcall.at[r]], buf_x, sem_gx).wait()
                    pltpu.async_copy(buf_x, acc.at[didx_all.at[r]], sem_sx,
                                     add=True)

                    @pl.when(r >= 1)
                    def _():
                        pltpu.make_async_copy(
                            buf_y, acc.at[didx_all.at[0]], sem_sy).wait()

                    @pl.when(r + 1 < NCH)
                    def _():
                        pltpu.async_copy(hp_hbm.at[srcall.at[r + 1]],
                                         buf_y, sem_gy)

                def chunk_body(r, _):
                    @pl.when(r % 2 == 0)
                    def _():
                        ping(r, buf_a, sem_a, sem_sa, buf_b, sem_b, sem_sb)

                    @pl.when(r % 2 == 1)
                    def _():
                        ping(r, buf_b, sem_b, sem_sb, buf_a, sem_a, sem_sa)
                    return 0
                lax.fori_loop(0, NCH, chunk_body, 0)
                pltpu.make_async_copy(
                    buf_b, acc.at[didx_all.at[0]], sem_sb).wait()
        plsc.subcore_barrier()

        @pl.when(s < 15)
        def _():
            pltpu.sync_copy(acc.at[pl.ds(s * 624, 624)],
                            out_hbm.at[p, pl.ds(s * 624, 624)])

        @pl.when(s == 15)
        def _():
            pltpu.sync_copy(acc.at[pl.ds(9360, 640)],
                            out_hbm.at[p, pl.ds(9360, 640)])
        plsc.subcore_barrier()


def _agg(hp, src2, dst2, nsl, ones_mode=False):
    return pl.kernel(
        functools.partial(_agg_body, nsl, ones_mode),
        out_type=jax.ShapeDtypeStruct((nsl, N_NODES, 128), jnp.float32),
        mesh=_mesh(),
        scratch_types=[
            pltpu.VMEM((CHUNK, 128), jnp.float32),
            pltpu.VMEM((CHUNK, 128), jnp.float32),
            pltpu.VMEM((16, 128), jnp.float32),
            pltpu.VMEM((NCH, CHUNK), jnp.int32),
            pltpu.VMEM((NCH, CHUNK), jnp.int32),
            pltpu.VMEM_SHARED((ACC_ROWS, 128), jnp.float32),
            pltpu.SemaphoreType.DMA,
            pltpu.SemaphoreType.DMA,
            pltpu.SemaphoreType.DMA,
            pltpu.SemaphoreType.DMA,
        ],
    )(hp, src2, dst2)


_MB = 1000


def _conv_kernel(leaky, nsl, h_ref, s_ref, degm_ref, wa_ref, wb_ref, b_ref,
                 o_ref):
    deg = jnp.maximum(degm_ref[0, :, :1], 1.0)
    acc = jnp.zeros((_MB, D_LAT), jnp.float32) + b_ref[...]
    for p in range(nsl):
        acc += jnp.dot(h_ref[p], wa_ref[pl.ds(p * 128, 128)],
                       preferred_element_type=jnp.float32)
        acc += jnp.dot(s_ref[p] / deg, wb_ref[pl.ds(p * 128, 128)],
                       preferred_element_type=jnp.float32)
    if leaky:
        acc = jnp.where(acc > 0, acc, 0.01 * acc)
    for p in range(D_LAT // 128):
        o_ref[p] = acc[:, p * 128:(p + 1) * 128]


def _conv_tc(ht, st, degm, W, bvec, leaky):
    nsl = ht.shape[0]
    d = nsl * 128
    wa, wb = W[:d], W[d:]
    grid = N_NODES // _MB
    return pl.pallas_call(
        functools.partial(_conv_kernel, leaky, nsl),
        grid=(grid,),
        in_specs=[
            pl.BlockSpec((nsl, _MB, 128), lambda i: (0, i, 0)),
            pl.BlockSpec((nsl, _MB, 128), lambda i: (0, i, 0)),
            pl.BlockSpec((1, _MB, 128), lambda i: (0, i, 0)),
            pl.BlockSpec((d, D_LAT), lambda i: (0, 0)),
            pl.BlockSpec((d, D_LAT), lambda i: (0, 0)),
            pl.BlockSpec((1, D_LAT), lambda i: (0, 0)),
        ],
        out_specs=pl.BlockSpec((D_LAT // 128, _MB, 128), lambda i: (0, i, 0)),
        out_shape=jax.ShapeDtypeStruct((D_LAT // 128, N_NODES, 128),
                                       jnp.float32),
        compiler_params=pltpu.CompilerParams(
            dimension_semantics=("arbitrary",)),
    )(ht, st, degm, wa, wb, bvec.reshape(1, D_LAT))


_FB = 2000
_SEG = N_NODES // BATCH


def _final_kernel(f_ref, wm_ref, bm_ref, ws_ref, bs_ref,
                  mean_ref, ls_ref, acc_ref):
    i = pl.program_id(0)

    @pl.when(i == 0)
    def _():
        acc_ref[...] = jnp.zeros_like(acc_ref)

    row = i * _FB + lax.broadcasted_iota(jnp.int32, (1, _FB), 1)
    seg = row // _SEG
    onehot = (lax.broadcasted_iota(jnp.int32, (BATCH, _FB), 0)
              == seg).astype(jnp.float32)
    pooled = jnp.concatenate(
        [jnp.dot(onehot, f_ref[p], preferred_element_type=jnp.float32)
         for p in range(D_LAT // 128)], axis=1)
    acc_ref[...] += pooled

    @pl.when(i == pl.num_programs(0) - 1)
    def _():
        tot = acc_ref[...]
        mu = jnp.mean(tot, axis=-1, keepdims=True)
        var = jnp.mean((tot - mu) ** 2, axis=-1, keepdims=True)
        hn = (tot - mu) / jnp.sqrt(var + 1e-5)
        mean_ref[...] = jnp.dot(hn, wm_ref[...],
                                preferred_element_type=jnp.float32) + bm_ref[...]
        ls_ref[...] = jnp.dot(hn, ws_ref[...],
                              preferred_element_type=jnp.float32) + bs_ref[...]


def _final_tc(featt, Wm, bm, Ws, bs):
    grid = N_NODES // _FB
    return pl.pallas_call(
        _final_kernel,
        grid=(grid,),
        in_specs=[
            pl.BlockSpec((D_LAT // 128, _FB, 128), lambda i: (0, i, 0)),
            pl.BlockSpec((D_LAT, D_LAT), lambda i: (0, 0)),
            pl.BlockSpec((1, D_LAT), lambda i: (0, 0)),
            pl.BlockSpec((D_LAT, D_LAT), lambda i: (0, 0)),
            pl.BlockSpec((1, D_LAT), lambda i: (0, 0)),
        ],
        out_specs=[
            pl.BlockSpec((BATCH, D_LAT), lambda i: (0, 0)),
            pl.BlockSpec((BATCH, D_LAT), lambda i: (0, 0)),
        ],
        out_shape=[
            jax.ShapeDtypeStruct((BATCH, D_LAT), jnp.float32),
            jax.ShapeDtypeStruct((BATCH, D_LAT), jnp.float32),
        ],
        scratch_shapes=[pltpu.VMEM((BATCH, D_LAT), jnp.float32)],
        compiler_params=pltpu.CompilerParams(
            dimension_semantics=("arbitrary",)),
    )(featt, Wm, bm.reshape(1, D_LAT), Ws, bs.reshape(1, D_LAT))


def kernel(features, edge_index, batchSize, W1, b1, W2, b2, Wf, bf,
           Wp, bp, Wm, bm, Ws, bs):
    src = edge_index[0]
    dst = edge_index[1]

    srcp = jnp.concatenate(
        [src.reshape(NW, EC),
         jnp.zeros((NW, ECP - EC), jnp.int32)], axis=1).reshape(-1, CHUNK)
    dstp = jnp.concatenate(
        [dst.reshape(NW, EC),
         jnp.full((NW, ECP - EC), DUMMY, jnp.int32)], axis=1).reshape(-1, CHUNK)

    degm = _agg(jnp.ones((8, 128), jnp.float32), srcp, dstp, 2,
                ones_mode=True)

    ft = features.reshape(N_NODES, 2, 128).transpose(1, 0, 2)
    s1 = _agg(ft.reshape(2 * N_NODES, 128), srcp, dstp, 2)
    h1 = _conv_tc(ft, s1, degm, W1, b1, leaky=True)

    s2 = _agg(h1.reshape(8 * N_NODES, 128), srcp, dstp, 8)
    h2 = _conv_tc(h1, s2, degm, W2, b2, leaky=False)

    s3 = _agg(h2.reshape(8 * N_NODES, 128), srcp, dstp, 8)
    feat = _conv_tc(h2, s3, degm, Wf, bf, leaky=True)

    return _final_tc(feat, Wm, bm, Ws, bs)


# speedup vs baseline: 1.0807x; 1.0807x over previous
import functools

import jax
import jax.numpy as jnp
from jax import lax
from jax.experimental import pallas as pl
from jax.experimental.pallas import tpu as pltpu
from jax.experimental.pallas import tpu_sc as plsc

N_NODES = 10000
N_EDGES = 160000
D_LAT = 1024
BATCH = 8

NW = 32
EC = N_EDGES // NW
CHUNK = 128
ECP = 5120
NCH = ECP // CHUNK
ACC_ROWS = 10016
DUMMY = N_NODES

_mesh = lambda: plsc.VectorSubcoreMesh(core_axis_name="c", subcore_axis_name="s")


NTOT = 2 * NCH


def _agg_body(nsl, nones, hp_hbm, src2_hbm, dst2_hbm, out_hbm,
              buf_a, buf_b, zbuf, srcall, didx_all, acc,
              sem_a, sem_b, sem_sa, sem_sb):
    c = lax.axis_index("c")
    s = lax.axis_index("s")

    for r in range(16):
        def zz(i, _):
            zbuf[r, pl.ds(i * 16, 16)] = jnp.zeros((16,), jnp.float32)
            return 0
        lax.fori_loop(0, 8, zz, 0)

    nzc = ACC_ROWS // 16
    for pp in range(nsl // 2):
        p = 2 * pp + c
        ones_mode = pp >= (nsl - nones) // 2
        if ones_mode:
            for r in range(CHUNK):
                def oo(i, _):
                    buf_a[r, pl.ds(i * 16, 16)] = (
                        jnp.ones((16,), jnp.float32))
                    return 0
                lax.fori_loop(0, 8, oo, 0)

        def zc(j, _):
            idx = s + j * 16

            @pl.when(idx < nzc)
            def _():
                pltpu.sync_copy(zbuf, acc.at[pl.ds(idx * 16, 16)])
            return 0
        lax.fori_loop(0, (nzc + 15) // 16, zc, 0)
        plsc.subcore_barrier()

        for half in range(2):
            eb = s + 16 * half
            pltpu.sync_copy(dst2_hbm.at[pl.ds(eb * NCH, NCH)], didx_all)
            if ones_mode:
                def chunk_body(r, _):
                    pltpu.sync_copy(buf_a, acc.at[didx_all.at[r]], add=True)
                    return 0
                lax.fori_loop(0, NCH, chunk_body, 0)
            else:
                pltpu.sync_copy(src2_hbm.at[pl.ds(eb * NCH, NCH)], srcall)
                off = p * N_NODES

                def ao(i, _):
                    srcall[i // 8, pl.ds((i % 8) * 16, 16)] = (
                        srcall[i // 8, pl.ds((i % 8) * 16, 16)] + off)
                    return 0
                lax.fori_loop(0, NCH * 8, ao, 0)

                pltpu.async_copy(hp_hbm.at[srcall.at[0]], buf_a, sem_a)

                def chunk_body(r, _):
                    nxt = r + 1

                    @pl.when(r % 2 == 0)
                    def _():
                        @pl.when(nxt < NCH)
                        def _():
                            pltpu.async_copy(hp_hbm.at[srcall.at[nxt]],
                                             buf_b, sem_b)
                        pltpu.make_async_copy(
                            hp_hbm.at[srcall.at[r]], buf_a, sem_a).wait()
                        pltpu.sync_copy(buf_a, acc.at[didx_all.at[r]],
                                        add=True)

                    @pl.when(r % 2 == 1)
                    def _():
                        @pl.when(nxt < NCH)
                        def _():
                            pltpu.async_copy(hp_hbm.at[srcall.at[nxt]],
                                             buf_a, sem_a)
                        pltpu.make_async_copy(
                            hp_hbm.at[srcall.at[r]], buf_b, sem_b).wait()
                        pltpu.sync_copy(buf_b, acc.at[didx_all.at[r]],
                                        add=True)
                    return 0
                lax.fori_loop(0, NCH, chunk_body, 0)
        plsc.subcore_barrier()

        @pl.when(s < 15)
        def _():
            pltpu.sync_copy(acc.at[pl.ds(s * 624, 624)],
                            out_hbm.at[p, pl.ds(s * 624, 624)])

        @pl.when(s == 15)
        def _():
            pltpu.sync_copy(acc.at[pl.ds(9360, 640)],
                            out_hbm.at[p, pl.ds(9360, 640)])
        plsc.subcore_barrier()


def _agg(hp, src2, dst2, nsl, nones=0):
    return pl.kernel(
        functools.partial(_agg_body, nsl, nones),
        out_type=jax.ShapeDtypeStruct((nsl, N_NODES, 128), jnp.float32),
        mesh=_mesh(),
        scratch_types=[
            pltpu.VMEM((CHUNK, 128), jnp.float32),
            pltpu.VMEM((CHUNK, 128), jnp.float32),
            pltpu.VMEM((16, 128), jnp.float32),
            pltpu.VMEM((NCH, CHUNK), jnp.int32),
            pltpu.VMEM((NCH, CHUNK), jnp.int32),
            pltpu.VMEM_SHARED((ACC_ROWS, 128), jnp.float32),
            pltpu.SemaphoreType.DMA,
            pltpu.SemaphoreType.DMA,
            pltpu.SemaphoreType.DMA,
            pltpu.SemaphoreType.DMA,
        ],
    )(hp, src2, dst2)


_MB = 1000


def _conv_kernel(leaky, nsl, h_ref, s_ref, degm_ref, wa_ref, wb_ref, b_ref,
                 o_ref):
    deg = jnp.maximum(degm_ref[0, :, :1].astype(jnp.float32), 1.0)
    acc = jnp.zeros((_MB, D_LAT), jnp.float32) + b_ref[...]
    for p in range(nsl):
        acc += jnp.dot(h_ref[p].astype(jnp.float32),
                       wa_ref[pl.ds(p * 128, 128)],
                       preferred_element_type=jnp.float32)
        acc += jnp.dot(s_ref[p].astype(jnp.float32) / deg,
                       wb_ref[pl.ds(p * 128, 128)],
                       preferred_element_type=jnp.float32)
    if leaky:
        acc = jnp.where(acc > 0, acc, 0.01 * acc)
    for p in range(D_LAT // 128):
        o_ref[p] = acc[:, p * 128:(p + 1) * 128]


def _conv_tc(ht, st, degm, W, bvec, leaky):
    nsl = ht.shape[0]
    d = nsl * 128
    wa, wb = W[:d], W[d:]
    grid = N_NODES // _MB
    return pl.pallas_call(
        functools.partial(_conv_kernel, leaky, nsl),
        grid=(grid,),
        in_specs=[
            pl.BlockSpec((nsl, _MB, 128), lambda i: (0, i, 0)),
            pl.BlockSpec((nsl, _MB, 128), lambda i: (0, i, 0)),
            pl.BlockSpec((1, _MB, 128), lambda i: (2, i, 0)),
            pl.BlockSpec((d, D_LAT), lambda i: (0, 0)),
            pl.BlockSpec((d, D_LAT), lambda i: (0, 0)),
            pl.BlockSpec((1, D_LAT), lambda i: (0, 0)),
        ],
        out_specs=pl.BlockSpec((D_LAT // 128, _MB, 128), lambda i: (0, i, 0)),
        out_shape=jax.ShapeDtypeStruct((D_LAT // 128, N_NODES, 128),
                                       jnp.float32),
        compiler_params=pltpu.CompilerParams(
            dimension_semantics=("arbitrary",)),
    )(ht, st, degm, wa, wb, bvec.reshape(1, D_LAT))


_FB = 2000
_SEG = N_NODES // BATCH


def _final_kernel(f_ref, wm_ref, bm_ref, ws_ref, bs_ref,
                  mean_ref, ls_ref, acc_ref):
    i = pl.program_id(0)

    @pl.when(i == 0)
    def _():
        acc_ref[...] = jnp.zeros_like(acc_ref)

    row = i * _FB + lax.broadcasted_iota(jnp.int32, (1, _FB), 1)
    seg = row // _SEG
    onehot = (lax.broadcasted_iota(jnp.int32, (BATCH, _FB), 0)
              == seg).astype(jnp.float32)
    pooled = jnp.concatenate(
        [jnp.dot(onehot, f_ref[p].astype(jnp.float32),
                 preferred_element_type=jnp.float32)
         for p in range(D_LAT // 128)], axis=1)
    acc_ref[...] += pooled

    @pl.when(i == pl.num_programs(0) - 1)
    def _():
        tot = acc_ref[...]
        mu = jnp.mean(tot, axis=-1, keepdims=True)
        var = jnp.mean((tot - mu) ** 2, axis=-1, keepdims=True)
        hn = (tot - mu) / jnp.sqrt(var + 1e-5)
        mean_ref[...] = jnp.dot(hn, wm_ref[...],
                                preferred_element_type=jnp.float32) + bm_ref[...]
        ls_ref[...] = jnp.dot(hn, ws_ref[...],
                              preferred_element_type=jnp.float32) + bs_ref[...]


def _final_tc(featt, Wm, bm, Ws, bs):
    grid = N_NODES // _FB
    return pl.pallas_call(
        _final_kernel,
        grid=(grid,),
        in_specs=[
            pl.BlockSpec((D_LAT // 128, _FB, 128), lambda i: (0, i, 0)),
            pl.BlockSpec((D_LAT, D_LAT), lambda i: (0, 0)),
            pl.BlockSpec((1, D_LAT), lambda i: (0, 0)),
            pl.BlockSpec((D_LAT, D_LAT), lambda i: (0, 0)),
            pl.BlockSpec((1, D_LAT), lambda i: (0, 0)),
        ],
        out_specs=[
            pl.BlockSpec((BATCH, D_LAT), lambda i: (0, 0)),
            pl.BlockSpec((BATCH, D_LAT), lambda i: (0, 0)),
        ],
        out_shape=[
            jax.ShapeDtypeStruct((BATCH, D_LAT), jnp.float32),
            jax.ShapeDtypeStruct((BATCH, D_LAT), jnp.float32),
        ],
        scratch_shapes=[pltpu.VMEM((BATCH, D_LAT), jnp.float32)],
        compiler_params=pltpu.CompilerParams(
            dimension_semantics=("arbitrary",)),
    )(featt, Wm, bm.reshape(1, D_LAT), Ws, bs.reshape(1, D_LAT))


def kernel(features, edge_index, batchSize, W1, b1, W2, b2, Wf, bf,
           Wp, bp, Wm, bm, Ws, bs):
    src = edge_index[0]
    dst = edge_index[1]

    srcp = jnp.concatenate(
        [src.reshape(NW, EC),
         jnp.zeros((NW, ECP - EC), jnp.int32)], axis=1).reshape(-1, CHUNK)
    dstp = jnp.concatenate(
        [dst.reshape(NW, EC),
         jnp.full((NW, ECP - EC), DUMMY, jnp.int32)], axis=1).reshape(-1, CHUNK)

    ft = features.reshape(N_NODES, 2, 128).transpose(1, 0, 2)
    s1d = _agg(ft.reshape(2 * N_NODES, 128), srcp, dstp, 4, nones=2)
    h1 = _conv_tc(ft, s1d, s1d, W1, b1, leaky=True)

    s2 = _agg(h1.reshape(8 * N_NODES, 128), srcp, dstp, 8)
    h2 = _conv_tc(h1, s2, s1d, W2, b2, leaky=False)

    s3 = _agg(h2.reshape(8 * N_NODES, 128), srcp, dstp, 8)
    feat = _conv_tc(h2, s3, s1d, Wf, bf, leaky=True)

    return _final_tc(feat, Wm, bm, Ws, bs)
